# baseline (speedup 1.0000x reference)
import functools

import jax
import jax.numpy as jnp
from jax import lax
from jax.experimental import pallas as pl
from jax.experimental.pallas import tpu as pltpu
from jax.experimental.pallas import tpu_sc as plsc

_B = 4
_N = 50000
_K = 32
_EMB = 8
_FEAT = 4
_SLOPE = 0.2
_BF = _B * _FEAT

_NC = 2
_NS = 16
_NW = _NC * _NS

_NB = 16
_NBLK = 98
_CH = _NB * _NBLK
_NPAD = _NW * _CH

_TB = _NPAD // 8


def _tc_body(e_ref, m_ref, o_ref):
    z = jnp.dot(e_ref[...], m_ref[...], preferred_element_type=jnp.float32)
    o_ref[...] = jnp.where(z >= 0, z, _SLOPE * z)


def _sc_body(f_hbm, nbr_hbm, w_hbm,
             idx0, idx1, g0, g1, wo0, wo1, gs0, gs1, os0, os1):
    idx = (idx0, idx1)
    gv = (g0, g1)
    wo = (wo0, wo1)
    gs = (gs0, gs1)
    osem = (os0, os1)
    wid = lax.axis_index("s") * _NC + lax.axis_index("c")
    base0 = wid * _CH
    i16 = lax.iota(jnp.int32, 16)

    def fill_and_fire(p, blk):
        base = base0 + blk * _NB
        pltpu.sync_copy(nbr_hbm.at[pl.ds(base * _K, _NB * _K)],
                        idx[p].at[pl.ds(0, _NB * _K)])
        idx[p][pl.ds(_NB * _K, _NB)] = base + i16
        pltpu.async_copy(f_hbm.at[idx[p]], gv[p], gs[p])

    fill_and_fire(0, 0)
    fill_and_fire(1, 1)

    def one(p, i):
        blk = 2 * i + p
        base = base0 + blk * _NB
        pltpu.make_async_copy(f_hbm.at[idx[p]], gv[p], gs[p]).wait()

        @pl.when(blk >= 2)
        def _():
            pltpu.make_async_copy(
                wo[p], w_hbm.at[pl.ds(0, _NB * _K)], osem[p]
            ).wait()

        cst = [plsc.load_gather(gv[p],
                                [i16 + (_NB * _K), jnp.full((16,), c, jnp.int32)])
               for c in range(_BF)]
        iK = i16 * _K

        @plsc.parallel_loop(0, _K, unroll=2)
        def kloop(k):
            rows = iK + k
            acc = [None] * _B
            for c in range(_BF):
                gc = plsc.load_gather(
                    gv[p], [rows, jnp.full((16,), c, jnp.int32)])
                d = cst[c] - gc
                s = d * d
                b = c // _FEAT
                acc[b] = s if acc[b] is None else acc[b] + s
            w16 = (jnp.exp(-acc[0]) + jnp.exp(-acc[1])
                   + jnp.exp(-acc[2]) + jnp.exp(-acc[3])) * (1.0 / _B)
            kidx = plsc.load_gather(idx[p], [rows])
            w16 = jnp.where(kidx != -1, w16, 0.0)
            plsc.store_scatter(wo[p], [rows], w16)

        pltpu.async_copy(wo[p], w_hbm.at[pl.ds(base * _K, _NB * _K)], osem[p])

        @pl.when(blk + 2 < _NBLK)
        def _():
            fill_and_fire(p, blk + 2)

    def iteration(i, carry):
        one(0, i)
        one(1, i)
        return carry

    lax.fori_loop(0, _NBLK // 2, iteration, 0)
    pltpu.make_async_copy(wo[0], w_hbm.at[pl.ds(0, _NB * _K)], osem[0]).wait()
    pltpu.make_async_copy(wo[1], w_hbm.at[pl.ds(0, _NB * _K)], osem[1]).wait()


_sc_call = functools.partial(
    pl.kernel,
    out_type=jax.ShapeDtypeStruct((_NPAD * _K,), jnp.float32),
    mesh=plsc.VectorSubcoreMesh(core_axis_name="c", subcore_axis_name="s"),
    scratch_types=[
        pltpu.VMEM((_NB * _K + _NB,), jnp.int32),
        pltpu.VMEM((_NB * _K + _NB,), jnp.int32),
        pltpu.VMEM((_NB * _K + _NB, _BF), jnp.float32),
        pltpu.VMEM((_NB * _K + _NB, _BF), jnp.float32),
        pltpu.VMEM((_NB * _K,), jnp.float32),
        pltpu.VMEM((_NB * _K,), jnp.float32),
        pltpu.SemaphoreType.DMA,
        pltpu.SemaphoreType.DMA,
        pltpu.SemaphoreType.DMA,
        pltpu.SemaphoreType.DMA,
    ],
    compiler_params=pltpu.CompilerParams(
        needs_layout_passes=False, use_tc_tiling_on_sc=False
    ),
)(_sc_body)


def kernel(x, node_embeddings, fc_weight, fc_bias, theta, neighbor_list):
    scale = lax.rsqrt(2.0 * theta.astype(jnp.float32))
    w0 = fc_weight[:, 0]
    we = fc_weight[:, 1:]
    m = jnp.concatenate(
        [
            jnp.kron(jnp.eye(_B, dtype=jnp.float32), w0[None, :]),
            jnp.tile(we.T, (1, _B)),
            jnp.tile(fc_bias, _B)[None, :],
            jnp.zeros((3, _BF), dtype=jnp.float32),
        ],
        axis=0,
    ) * scale
    e = jnp.concatenate(
        [
            x.T,
            node_embeddings,
            jnp.ones((_N, 1), dtype=jnp.float32),
            jnp.zeros((_N, 3), dtype=jnp.float32),
        ],
        axis=1,
    )
    e = jnp.pad(e, ((0, _NPAD - _N), (0, 0)))

    f_rows = pl.pallas_call(
        _tc_body,
        grid=(_NPAD // _TB,),
        in_specs=[
            pl.BlockSpec((_TB, _BF), lambda i: (i, 0)),
            pl.BlockSpec((_BF, _BF), lambda i: (0, 0)),
        ],
        out_specs=pl.BlockSpec((_TB, _BF), lambda i: (i, 0)),
        out_shape=jax.ShapeDtypeStruct((_NPAD, _BF), jnp.float32),
    )(e, m)

    nbr = jnp.pad(neighbor_list, ((0, _NPAD - _N), (0, 0))).reshape(-1)
    w_full = _sc_call(f_rows, nbr)
    return w_full.reshape(_NPAD, _K)[:_N]


# speedup vs baseline: 1.5058x; 1.5058x over previous
import functools

import jax
import jax.numpy as jnp
from jax import lax
from jax.experimental import pallas as pl
from jax.experimental.pallas import tpu as pltpu
from jax.experimental.pallas import tpu_sc as plsc

_B = 4
_N = 50000
_K = 32
_EMB = 8
_FEAT = 4
_SLOPE = 0.2
_BF = _B * _FEAT

_NC = 2
_NS = 16
_NW = _NC * _NS

_NB = 16
_NBLK = 98
_CH = _NB * _NBLK
_NPAD = _NW * _CH

_TB = _NPAD // 8


def _tc_body(e_ref, m_ref, o_ref):
    z = jnp.dot(e_ref[...], m_ref[...], preferred_element_type=jnp.float32)
    o_ref[...] = jnp.where(z >= 0, z, _SLOPE * z)


def _sc_body(f_hbm, nbr_hbm, w_hbm,
             idx0, idx1, g0, g1, wo0, wo1, gs0, gs1, os0, os1):
    idx = (idx0, idx1)
    gv = (g0, g1)
    wo = (wo0, wo1)
    gs = (gs0, gs1)
    osem = (os0, os1)
    wid = lax.axis_index("s") * _NC + lax.axis_index("c")
    base0 = wid * _CH
    i16 = lax.iota(jnp.int32, 16)

    def fill_and_fire(p, blk):
        base = base0 + blk * _NB
        pltpu.sync_copy(nbr_hbm.at[pl.ds(base * _K, _NB * _K)],
                        idx[p].at[pl.ds(0, _NB * _K)])
        idx[p][pl.ds(_NB * _K, _NB)] = base + i16
        pltpu.async_copy(f_hbm.at[idx[p]], gv[p], gs[p])

    fill_and_fire(0, 0)
    fill_and_fire(1, 1)

    def one(p, i):
        blk = 2 * i + p
        base = base0 + blk * _NB
        pltpu.make_async_copy(f_hbm.at[idx[p]], gv[p], gs[p]).wait()

        @pl.when(blk >= 2)
        def _():
            pltpu.make_async_copy(
                wo[p], w_hbm.at[pl.ds(0, _NB * _K)], osem[p]
            ).wait()

        @plsc.parallel_loop(0, _NB, unroll=2)
        def node(j):
            jr = jnp.full((16,), _NB * _K + j, jnp.int32)
            rows0 = i16 + (j * _K)
            rows1 = i16 + (j * _K + 16)
            acc = [None] * (2 * _B)
            for c in range(_BF):
                cs = plsc.load_gather(gv[p], [jr, jnp.full((16,), c, jnp.int32)])
                b = c // _FEAT
                for h in range(2):
                    gc = plsc.load_gather(
                        gv[p],
                        [rows0 if h == 0 else rows1,
                         jnp.full((16,), c, jnp.int32)])
                    d = cs - gc
                    s = d * d
                    a = 2 * b + h
                    acc[a] = s if acc[a] is None else acc[a] + s
            for h in range(2):
                w16 = (jnp.exp(-acc[h]) + jnp.exp(-acc[2 + h])
                       + jnp.exp(-acc[4 + h]) + jnp.exp(-acc[6 + h])) * (1.0 / _B)
                krow = idx[p][pl.ds(j * _K + h * 16, 16)]
                w16 = jnp.where(krow != -1, w16, 0.0)
                wo[p][pl.ds(j * _K + h * 16, 16)] = w16

        pltpu.async_copy(wo[p], w_hbm.at[pl.ds(base * _K, _NB * _K)], osem[p])

        @pl.when(blk + 2 < _NBLK)
        def _():
            fill_and_fire(p, blk + 2)

    def iteration(i, carry):
        one(0, i)
        one(1, i)
        return carry

    lax.fori_loop(0, _NBLK // 2, iteration, 0)
    pltpu.make_async_copy(wo[0], w_hbm.at[pl.ds(0, _NB * _K)], osem[0]).wait()
    pltpu.make_async_copy(wo[1], w_hbm.at[pl.ds(0, _NB * _K)], osem[1]).wait()


_sc_call = functools.partial(
    pl.kernel,
    out_type=jax.ShapeDtypeStruct((_NPAD * _K,), jnp.float32),
    mesh=plsc.VectorSubcoreMesh(core_axis_name="c", subcore_axis_name="s"),
    scratch_types=[
        pltpu.VMEM((_NB * _K + _NB,), jnp.int32),
        pltpu.VMEM((_NB * _K + _NB,), jnp.int32),
        pltpu.VMEM((_NB * _K + _NB, _BF), jnp.float32),
        pltpu.VMEM((_NB * _K + _NB, _BF), jnp.float32),
        pltpu.VMEM((_NB * _K,), jnp.float32),
        pltpu.VMEM((_NB * _K,), jnp.float32),
        pltpu.SemaphoreType.DMA,
        pltpu.SemaphoreType.DMA,
        pltpu.SemaphoreType.DMA,
        pltpu.SemaphoreType.DMA,
    ],
    compiler_params=pltpu.CompilerParams(
        needs_layout_passes=False, use_tc_tiling_on_sc=False
    ),
)(_sc_body)


def kernel(x, node_embeddings, fc_weight, fc_bias, theta, neighbor_list):
    scale = lax.rsqrt(2.0 * theta.astype(jnp.float32))
    w0 = fc_weight[:, 0]
    we = fc_weight[:, 1:]
    m = jnp.concatenate(
        [
            jnp.kron(jnp.eye(_B, dtype=jnp.float32), w0[None, :]),
            jnp.tile(we.T, (1, _B)),
            jnp.tile(fc_bias, _B)[None, :],
            jnp.zeros((3, _BF), dtype=jnp.float32),
        ],
        axis=0,
    ) * scale
    e = jnp.concatenate(
        [
            x.T,
            node_embeddings,
            jnp.ones((_N, 1), dtype=jnp.float32),
            jnp.zeros((_N, 3), dtype=jnp.float32),
        ],
        axis=1,
    )
    e = jnp.pad(e, ((0, _NPAD - _N), (0, 0)))

    f_rows = pl.pallas_call(
        _tc_body,
        grid=(_NPAD // _TB,),
        in_specs=[
            pl.BlockSpec((_TB, _BF), lambda i: (i, 0)),
            pl.BlockSpec((_BF, _BF), lambda i: (0, 0)),
        ],
        out_specs=pl.BlockSpec((_TB, _BF), lambda i: (i, 0)),
        out_shape=jax.ShapeDtypeStruct((_NPAD, _BF), jnp.float32),
    )(e, m)

    nbr = jnp.pad(neighbor_list, ((0, _NPAD - _N), (0, 0))).reshape(-1)
    w_full = _sc_call(f_rows, nbr)
    return w_full.reshape(_NPAD, _K)[:_N]


# speedup vs baseline: 1.7669x; 1.1734x over previous
import functools

import jax
import jax.numpy as jnp
from jax import lax
from jax.experimental import pallas as pl
from jax.experimental.pallas import tpu as pltpu
from jax.experimental.pallas import tpu_sc as plsc

_B = 4
_N = 50000
_K = 32
_EMB = 8
_FEAT = 4
_SLOPE = 0.2
_BF = _B * _FEAT

_NC = 2
_NS = 16
_NW = _NC * _NS

_NB = 16
_NBLK = 98
_NBLK_LAST = 87
_CH = _NB * _NBLK

_TB = _N // 5


def _tc_body(e_ref, m_ref, o_ref):
    z = jnp.dot(e_ref[...], m_ref[...], preferred_element_type=jnp.float32)
    o_ref[...] = jnp.where(z >= 0, z, _SLOPE * z)


def _sc_body(f_hbm, nbr_hbm, w_hbm,
             idx0, idx1, g0, g1, wo0, wo1, gs0, gs1, os0, os1):
    idx = (idx0, idx1)
    gv = (g0, g1)
    wo = (wo0, wo1)
    gs = (gs0, gs1)
    osem = (os0, os1)
    wid = lax.axis_index("s") * _NC + lax.axis_index("c")
    base0 = wid * _CH
    nblk = jnp.where(wid == _NW - 1, _NBLK_LAST, _NBLK)
    i16 = lax.iota(jnp.int32, 16)

    def fill_and_fire(p, blk):
        base = base0 + blk * _NB
        pltpu.sync_copy(nbr_hbm.at[pl.ds(base * _K, _NB * _K)],
                        idx[p].at[pl.ds(0, _NB * _K)])
        idx[p][pl.ds(_NB * _K, _NB)] = base + i16
        pltpu.async_copy(f_hbm.at[idx[p]], gv[p], gs[p])

    fill_and_fire(0, 0)
    fill_and_fire(1, 1)

    def one(p, i):
        blk = 2 * i + p

        @pl.when(blk < nblk)
        def _guarded():
            _one_body(p, blk)

    def _one_body(p, blk):
        base = base0 + blk * _NB
        pltpu.make_async_copy(f_hbm.at[idx[p]], gv[p], gs[p]).wait()

        @pl.when(blk >= 2)
        def _():
            pltpu.make_async_copy(
                wo[p], w_hbm.at[pl.ds(0, _NB * _K)], osem[p]
            ).wait()

        @plsc.parallel_loop(0, _NB, unroll=2)
        def node(j):
            jr = jnp.full((16,), _NB * _K + j, jnp.int32)
            rows0 = i16 + (j * _K)
            rows1 = i16 + (j * _K + 16)
            acc = [None] * (2 * _B)
            for c in range(_BF):
                cs = plsc.load_gather(gv[p], [jr, jnp.full((16,), c, jnp.int32)])
                b = c // _FEAT
                for h in range(2):
                    gc = plsc.load_gather(
                        gv[p],
                        [rows0 if h == 0 else rows1,
                         jnp.full((16,), c, jnp.int32)])
                    d = cs - gc
                    s = d * d
                    a = 2 * b + h
                    acc[a] = s if acc[a] is None else acc[a] + s
            for h in range(2):
                w16 = (jnp.exp(-acc[h]) + jnp.exp(-acc[2 + h])
                       + jnp.exp(-acc[4 + h]) + jnp.exp(-acc[6 + h])) * (1.0 / _B)
                krow = idx[p][pl.ds(j * _K + h * 16, 16)]
                w16 = jnp.where(krow != -1, w16, 0.0)
                wo[p][pl.ds(j * _K + h * 16, 16)] = w16

        pltpu.async_copy(wo[p], w_hbm.at[pl.ds(base * _K, _NB * _K)], osem[p])

        @pl.when(blk + 2 < nblk)
        def _():
            fill_and_fire(p, blk + 2)

    def iteration(i, carry):
        one(0, i)
        one(1, i)
        return carry

    lax.fori_loop(0, _NBLK // 2, iteration, 0)
    pltpu.make_async_copy(wo[0], w_hbm.at[pl.ds(0, _NB * _K)], osem[0]).wait()
    pltpu.make_async_copy(wo[1], w_hbm.at[pl.ds(0, _NB * _K)], osem[1]).wait()


_sc_call = functools.partial(
    pl.kernel,
    out_type=jax.ShapeDtypeStruct((_N * _K,), jnp.float32),
    mesh=plsc.VectorSubcoreMesh(core_axis_name="c", subcore_axis_name="s"),
    scratch_types=[
        pltpu.VMEM((_NB * _K + _NB,), jnp.int32),
        pltpu.VMEM((_NB * _K + _NB,), jnp.int32),
        pltpu.VMEM((_NB * _K + _NB, _BF), jnp.float32),
        pltpu.VMEM((_NB * _K + _NB, _BF), jnp.float32),
        pltpu.VMEM((_NB * _K,), jnp.float32),
        pltpu.VMEM((_NB * _K,), jnp.float32),
        pltpu.SemaphoreType.DMA,
        pltpu.SemaphoreType.DMA,
        pltpu.SemaphoreType.DMA,
        pltpu.SemaphoreType.DMA,
    ],
    compiler_params=pltpu.CompilerParams(
        needs_layout_passes=False, use_tc_tiling_on_sc=False
    ),
)(_sc_body)


def kernel(x, node_embeddings, fc_weight, fc_bias, theta, neighbor_list):
    scale = lax.rsqrt(2.0 * theta.astype(jnp.float32))
    w0 = fc_weight[:, 0]
    we = fc_weight[:, 1:]
    m = jnp.concatenate(
        [
            jnp.kron(jnp.eye(_B, dtype=jnp.float32), w0[None, :]),
            jnp.tile(we.T, (1, _B)),
            jnp.tile(fc_bias, _B)[None, :],
            jnp.zeros((3, _BF), dtype=jnp.float32),
        ],
        axis=0,
    ) * scale
    e = jnp.concatenate(
        [
            x.T,
            node_embeddings,
            jnp.ones((_N, 1), dtype=jnp.float32),
            jnp.zeros((_N, 3), dtype=jnp.float32),
        ],
        axis=1,
    )
    f_rows = pl.pallas_call(
        _tc_body,
        grid=(_N // _TB,),
        in_specs=[
            pl.BlockSpec((_TB, _BF), lambda i: (i, 0)),
            pl.BlockSpec((_BF, _BF), lambda i: (0, 0)),
        ],
        out_specs=pl.BlockSpec((_TB, _BF), lambda i: (i, 0)),
        out_shape=jax.ShapeDtypeStruct((_N, _BF), jnp.float32),
    )(e, m)

    w_full = _sc_call(f_rows, neighbor_list.reshape(-1))
    return w_full.reshape(_N, _K)
